# Initial kernel scaffold; baseline (speedup 1.0000x reference)
import jax
import jax.numpy as jnp
from jax.experimental import pallas as pl


def kernel(tutor_idx, time_idx, experience, subject_pca, grade_pca, tutor_table, time_table, Ws, bs, Wg, bg, We, be, W1, b1, W2, b2, W3, b3):
    raise NotImplementedError("write your pallas kernel here")


# speedup vs baseline: 1.5990x; 1.5990x over previous
import functools

import jax
import jax.numpy as jnp
from jax import lax
from jax.experimental import pallas as pl
from jax.experimental.pallas import tpu as pltpu
from jax.experimental.pallas import tpu_sc as plsc

_NC = 2
_NS = 16
_CHUNK = 128


def _sc_gather(tutor_idx, time_idx, tutor_table, time_table):
    B = tutor_idx.shape[0]
    E = tutor_table.shape[1]
    nw = _NC * _NS
    bpw = B // nw
    nch = bpw // _CHUNK

    tidx3 = tutor_idx.reshape(nw, nch, _CHUNK)
    midx3 = time_idx.reshape(nw, nch, _CHUNK)

    mesh = plsc.VectorSubcoreMesh(
        core_axis_name="c", subcore_axis_name="s",
        num_cores=_NC, num_subcores=_NS)

    @functools.partial(
        pl.kernel,
        mesh=mesh,
        compiler_params=pltpu.CompilerParams(use_tc_tiling_on_sc=False),
        out_type=(jax.ShapeDtypeStruct((B, E), jnp.float32),
                  jax.ShapeDtypeStruct((B, E), jnp.float32)),
        scratch_types=[
            pltpu.VMEM((nch, _CHUNK), jnp.int32),
            pltpu.VMEM((nch, _CHUNK), jnp.int32),
            pltpu.VMEM((bpw, E), jnp.float32),
            pltpu.VMEM((bpw, E), jnp.float32),
            pltpu.SemaphoreType.DMA,
            pltpu.SemaphoreType.DMA,
        ],
    )
    def gather_kernel(tidx_hbm, midx_hbm, ttab_hbm, mtab_hbm,
                      tout_hbm, mout_hbm,
                      tidx_v, midx_v, trows_v, mrows_v, tsem, msem):
        wid = lax.axis_index("s") * _NC + lax.axis_index("c")
        base = wid * bpw
        pltpu.sync_copy(tidx_hbm.at[wid], tidx_v)
        pltpu.sync_copy(midx_hbm.at[wid], midx_v)
        tcopies = []
        mcopies = []
        for j in range(nch):
            dst = pl.ds(j * _CHUNK, _CHUNK)
            tcopies.append(
                pltpu.async_copy(ttab_hbm.at[tidx_v.at[j]], trows_v.at[dst], tsem))
            mcopies.append(
                pltpu.async_copy(mtab_hbm.at[midx_v.at[j]], mrows_v.at[dst], msem))
        for c in tcopies:
            c.wait()
        for c in mcopies:
            c.wait()
        pltpu.sync_copy(trows_v, tout_hbm.at[pl.ds(base, bpw)])
        pltpu.sync_copy(mrows_v, mout_hbm.at[pl.ds(base, bpw)])

    return gather_kernel(tidx3, midx3, tutor_table, time_table)


def _mlp_body(tut, tim, feat, wblk, bsml, w1, b1, w2, b2, w3, b3, out):
    f32 = jnp.float32
    small = jnp.dot(feat[...], wblk[...], preferred_element_type=f32) + bsml[...]
    h = (jnp.dot(tut[...], w1[0:64, :], preferred_element_type=f32)
         + jnp.dot(tim[...], w1[64:128, :], preferred_element_type=f32)
         + jnp.dot(small, w1[128:224, :], preferred_element_type=f32)
         + b1[...])
    h = jnp.maximum(h, 0.0)
    h = jnp.maximum(jnp.dot(h, w2[...], preferred_element_type=f32) + b2[...], 0.0)
    out[...] = jnp.dot(h, w3[...], preferred_element_type=f32) + b3[...]


def _mlp(tut_emb, tim_emb, feat, wblk, bsml, W1, b1, W2, b2, W3, b3, bm=2048):
    B = tut_emb.shape[0]
    grid = (B // bm,)

    def bspec(shape, blocked):
        if blocked:
            return pl.BlockSpec((bm,) + shape[1:], lambda i: (i,) + (0,) * (len(shape) - 1))
        return pl.BlockSpec(shape, lambda i: (0,) * len(shape))

    in_specs = [
        bspec(tut_emb.shape, True),
        bspec(tim_emb.shape, True),
        bspec(feat.shape, True),
        bspec(wblk.shape, False),
        bspec(bsml.shape, False),
        bspec(W1.shape, False),
        bspec(b1.shape, False),
        bspec(W2.shape, False),
        bspec(b2.shape, False),
        bspec(W3.shape, False),
        bspec(b3.shape, False),
    ]
    return pl.pallas_call(
        _mlp_body,
        grid=grid,
        in_specs=in_specs,
        out_specs=pl.BlockSpec((bm, W3.shape[1]), lambda i: (i, 0)),
        out_shape=jax.ShapeDtypeStruct((B, W3.shape[1]), jnp.float32),
        compiler_params=pltpu.CompilerParams(
            dimension_semantics=("arbitrary",)),
    )(tut_emb, tim_emb, feat, wblk, bsml, W1, b1, W2, b2, W3, b3)


def kernel(tutor_idx, time_idx, experience, subject_pca, grade_pca,
           tutor_table, time_table, Ws, bs, Wg, bg, We, be,
           W1, b1, W2, b2, W3, b3):
    tut_emb, tim_emb = _sc_gather(tutor_idx, time_idx, tutor_table, time_table)

    feat = jnp.concatenate(
        [subject_pca, grade_pca, experience[:, None]], axis=1)
    z = jnp.zeros
    f32 = jnp.float32
    wblk = jnp.concatenate([
        jnp.concatenate([Ws, z((10, 64), f32)], axis=1),
        jnp.concatenate([z((5, 32), f32), Wg, z((5, 32), f32)], axis=1),
        jnp.concatenate([z((1, 64), f32), We], axis=1),
    ], axis=0)
    bsml = jnp.concatenate([bs, bg, be])[None, :]

    return _mlp(tut_emb, tim_emb, feat, wblk, bsml,
                W1, b1[None, :], W2, b2[None, :], W3, b3[None, :])


# speedup vs baseline: 1.7898x; 1.1193x over previous
import functools

import jax
import jax.numpy as jnp
from jax import lax
from jax.experimental import pallas as pl
from jax.experimental.pallas import tpu as pltpu
from jax.experimental.pallas import tpu_sc as plsc

_NC = 2
_NS = 16
_CHUNK = 128


def _sc_gather(idx2, tutor_table, time_table, B, E):
    nw = _NC * _NS
    bpw = B // nw
    nch = bpw // _CHUNK
    nrows = B // _CHUNK

    mesh = plsc.VectorSubcoreMesh(
        core_axis_name="c", subcore_axis_name="s",
        num_cores=_NC, num_subcores=_NS)

    @functools.partial(
        pl.kernel,
        mesh=mesh,
        compiler_params=pltpu.CompilerParams(use_tc_tiling_on_sc=False),
        out_type=jax.ShapeDtypeStruct((B, 2 * E), jnp.float32),
        scratch_types=[
            pltpu.VMEM((nch, _CHUNK), jnp.int32),
            pltpu.VMEM((nch, _CHUNK), jnp.int32),
            pltpu.VMEM((bpw, E), jnp.float32),
            pltpu.VMEM((bpw, E), jnp.float32),
            pltpu.SemaphoreType.DMA,
            pltpu.SemaphoreType.DMA,
        ],
    )
    def gather_kernel(idx_hbm, ttab_hbm, mtab_hbm, out_hbm,
                      tidx_v, midx_v, trows_v, mrows_v, tsem, msem):
        wid = lax.axis_index("s") * _NC + lax.axis_index("c")
        base = wid * bpw
        pltpu.sync_copy(idx_hbm.at[pl.ds(wid * nch, nch)], tidx_v)
        pltpu.sync_copy(idx_hbm.at[pl.ds(nrows + wid * nch, nch)], midx_v)
        tcopies = []
        mcopies = []
        for j in range(nch):
            dst = pl.ds(j * _CHUNK, _CHUNK)
            tcopies.append(
                pltpu.async_copy(ttab_hbm.at[tidx_v.at[j]], trows_v.at[dst], tsem))
            mcopies.append(
                pltpu.async_copy(mtab_hbm.at[midx_v.at[j]], mrows_v.at[dst], msem))
        for c in tcopies:
            c.wait()
        for c in mcopies:
            c.wait()
        rows = pl.ds(base, bpw)
        pltpu.sync_copy(trows_v, out_hbm.at[rows, pl.ds(0, E)])
        pltpu.sync_copy(mrows_v, out_hbm.at[rows, pl.ds(E, E)])

    return gather_kernel(idx2, tutor_table, time_table)


def _mlp_body(emb, feat, wblk, bsml, w1, b1, w2, b2, w3, b3, out):
    f32 = jnp.float32
    small = jnp.dot(feat[...], wblk[...], preferred_element_type=f32) + bsml[...]
    h = (jnp.dot(emb[...], w1[0:128, :], preferred_element_type=f32)
         + jnp.dot(small, w1[128:224, :], preferred_element_type=f32)
         + b1[...])
    h = jnp.maximum(h, 0.0)
    h = jnp.maximum(jnp.dot(h, w2[...], preferred_element_type=f32) + b2[...], 0.0)
    out[...] = jnp.dot(h, w3[...], preferred_element_type=f32) + b3[...]


def _mlp(emb, feat, wblk, bsml, W1, b1, W2, b2, W3, b3, bm=2048):
    B = emb.shape[0]
    grid = (B // bm,)

    def bspec(shape, blocked):
        if blocked:
            return pl.BlockSpec((bm,) + shape[1:], lambda i: (i,) + (0,) * (len(shape) - 1))
        return pl.BlockSpec(shape, lambda i: (0,) * len(shape))

    in_specs = [
        bspec(emb.shape, True),
        bspec(feat.shape, True),
        bspec(wblk.shape, False),
        bspec(bsml.shape, False),
        bspec(W1.shape, False),
        bspec(b1.shape, False),
        bspec(W2.shape, False),
        bspec(b2.shape, False),
        bspec(W3.shape, False),
        bspec(b3.shape, False),
    ]
    return pl.pallas_call(
        _mlp_body,
        grid=grid,
        in_specs=in_specs,
        out_specs=pl.BlockSpec((bm, W3.shape[1]), lambda i: (i, 0)),
        out_shape=jax.ShapeDtypeStruct((B, W3.shape[1]), jnp.float32),
        compiler_params=pltpu.CompilerParams(
            dimension_semantics=("arbitrary",)),
    )(emb, feat, wblk, bsml, W1, b1, W2, b2, W3, b3)


def kernel(tutor_idx, time_idx, experience, subject_pca, grade_pca,
           tutor_table, time_table, Ws, bs, Wg, bg, We, be,
           W1, b1, W2, b2, W3, b3):
    B = tutor_idx.shape[0]
    E = tutor_table.shape[1]

    idx2 = jnp.concatenate([tutor_idx, time_idx]).reshape(2 * B // _CHUNK, _CHUNK)
    emb = _sc_gather(idx2, tutor_table, time_table, B, E)

    feat = jnp.concatenate(
        [subject_pca, grade_pca, experience[:, None]], axis=1)
    z = jnp.zeros
    f32 = jnp.float32
    wblk = jnp.concatenate([
        jnp.concatenate([Ws, z((10, 64), f32)], axis=1),
        jnp.concatenate([z((5, 32), f32), Wg, z((5, 32), f32)], axis=1),
        jnp.concatenate([z((1, 64), f32), We], axis=1),
    ], axis=0)
    bsml = jnp.concatenate([bs, bg, be])[None, :]

    return _mlp(emb, feat, wblk, bsml,
                W1, b1[None, :], W2, b2[None, :], W3, b3[None, :])


# speedup vs baseline: 1.9370x; 1.0822x over previous
import functools

import jax
import jax.numpy as jnp
from jax import lax
from jax.experimental import pallas as pl
from jax.experimental.pallas import tpu as pltpu
from jax.experimental.pallas import tpu_sc as plsc

_NC = 2
_NS = 16
_CHUNK = 128


def _sc_gather(idx2, tutor_pad, time_pad, B, E):
    nw = _NC * _NS
    bpw = B // nw
    nch = bpw // _CHUNK
    nrows = B // _CHUNK

    mesh = plsc.VectorSubcoreMesh(
        core_axis_name="c", subcore_axis_name="s",
        num_cores=_NC, num_subcores=_NS)

    @functools.partial(
        pl.kernel,
        mesh=mesh,
        compiler_params=pltpu.CompilerParams(use_tc_tiling_on_sc=True),
        out_type=jax.ShapeDtypeStruct((B, 4 * E), jnp.float32),
        scratch_types=[
            pltpu.VMEM((nch, _CHUNK), jnp.int32),
            pltpu.VMEM((nch, _CHUNK), jnp.int32),
            pltpu.VMEM((bpw, 2 * E), jnp.float32),
            pltpu.VMEM((bpw // 2, 2 * E), jnp.float32),
            pltpu.SemaphoreType.DMA,
            pltpu.SemaphoreType.DMA,
        ],
    )
    def gather_kernel(idx_hbm, ttab_hbm, mtab_hbm, out_hbm,
                      tidx_v, midx_v, trows_v, mrows_v, tsem, msem):
        wid = lax.axis_index("s") * _NC + lax.axis_index("c")
        base = wid * bpw
        half = bpw // 2
        lanes_t = pl.ds(0, 2 * E)
        lanes_m = pl.ds(2 * E, 2 * E)
        pltpu.sync_copy(idx_hbm.at[pl.ds(wid * nch, nch)], tidx_v)
        pltpu.sync_copy(idx_hbm.at[pl.ds(nrows + wid * nch, nch)], midx_v)
        tcopies = []
        for j in range(nch):
            tcopies.append(pltpu.async_copy(
                ttab_hbm.at[tidx_v.at[j]], trows_v.at[pl.ds(j * _CHUNK, _CHUNK)],
                tsem))
        for p in range(2):
            mcopies = []
            for j in range(nch // 2):
                mcopies.append(pltpu.async_copy(
                    mtab_hbm.at[midx_v.at[p * (nch // 2) + j]],
                    mrows_v.at[pl.ds(j * _CHUNK, _CHUNK)], msem))
            for c in mcopies:
                c.wait()
            pltpu.sync_copy(mrows_v, out_hbm.at[pl.ds(base + p * half, half), lanes_m])
        for c in tcopies:
            c.wait()
        pltpu.sync_copy(trows_v, out_hbm.at[pl.ds(base, bpw), lanes_t])

    return gather_kernel(idx2, tutor_pad, time_pad)


def _mlp_body(emb, featT, w1ab, wblk, bsml, w1, b1, w2, b2, w3, b3, outT):
    f32 = jnp.float32
    small = lax.dot_general(
        featT[...], wblk[...], (((0,), (0,)), ((), ())),
        preferred_element_type=f32) + bsml[...]
    h = (jnp.dot(emb[...], w1ab[...], preferred_element_type=f32)
         + jnp.dot(small, w1[128:224, :], preferred_element_type=f32)
         + b1[...])
    h = jnp.maximum(h, 0.0)
    h = jnp.maximum(jnp.dot(h, w2[...], preferred_element_type=f32) + b2[...], 0.0)
    out = jnp.dot(h, w3[...], preferred_element_type=f32) + b3[...]
    outT[...] = out.T


def _mlp(emb, featT, w1ab, wblk, bsml, W1, b1, W2, b2, W3, b3, bm=2048):
    B = emb.shape[0]
    grid = (B // bm,)
    no = W3.shape[1]

    in_specs = [
        pl.BlockSpec((bm, emb.shape[1]), lambda i: (i, 0)),
        pl.BlockSpec((featT.shape[0], bm), lambda i: (0, i)),
        pl.BlockSpec(w1ab.shape, lambda i: (0, 0)),
        pl.BlockSpec(wblk.shape, lambda i: (0, 0)),
        pl.BlockSpec(bsml.shape, lambda i: (0, 0)),
        pl.BlockSpec(W1.shape, lambda i: (0, 0)),
        pl.BlockSpec(b1.shape, lambda i: (0, 0)),
        pl.BlockSpec(W2.shape, lambda i: (0, 0)),
        pl.BlockSpec(b2.shape, lambda i: (0, 0)),
        pl.BlockSpec(W3.shape, lambda i: (0, 0)),
        pl.BlockSpec(b3.shape, lambda i: (0, 0)),
    ]
    return pl.pallas_call(
        _mlp_body,
        grid=grid,
        in_specs=in_specs,
        out_specs=pl.BlockSpec((no, bm), lambda i: (0, i)),
        out_shape=jax.ShapeDtypeStruct((no, B), jnp.float32),
        compiler_params=pltpu.CompilerParams(
            dimension_semantics=("arbitrary",)),
    )(emb, featT, w1ab, wblk, bsml, W1, b1, W2, b2, W3, b3)


def kernel(tutor_idx, time_idx, experience, subject_pca, grade_pca,
           tutor_table, time_table, Ws, bs, Wg, bg, We, be,
           W1, b1, W2, b2, W3, b3):
    B = tutor_idx.shape[0]
    E = tutor_table.shape[1]

    idx2 = jnp.concatenate([tutor_idx, time_idx]).reshape(2 * B // _CHUNK, _CHUNK)
    tutor_pad = jnp.pad(tutor_table, ((0, 0), (0, E)))
    time_pad = jnp.pad(time_table, ((0, 0), (0, E)))
    emb = _sc_gather(idx2, tutor_pad, time_pad, B, E)

    featT = jnp.concatenate(
        [subject_pca.T, grade_pca.T, experience[None, :]], axis=0)
    z = jnp.zeros
    f32 = jnp.float32
    wblk = jnp.concatenate([
        jnp.concatenate([Ws, z((10, 64), f32)], axis=1),
        jnp.concatenate([z((5, 32), f32), Wg, z((5, 32), f32)], axis=1),
        jnp.concatenate([z((1, 64), f32), We], axis=1),
    ], axis=0)
    bsml = jnp.concatenate([bs, bg, be])[None, :]
    w1ab = jnp.concatenate([
        W1[0:E, :], z((E, W1.shape[1]), f32),
        W1[E:2 * E, :], z((E, W1.shape[1]), f32),
    ], axis=0)

    outT = _mlp(emb, featT, w1ab, wblk, bsml,
                W1, b1[None, :], W2, b2[None, :], W3, b3[None, :])
    return outT.T


# speedup vs baseline: 2.0495x; 1.0581x over previous
import functools

import jax
import jax.numpy as jnp
from jax import lax
from jax.experimental import pallas as pl
from jax.experimental.pallas import tpu as pltpu
from jax.experimental.pallas import tpu_sc as plsc

_NC = 2
_NS = 16
_CHUNK = 128


def _sc_gather(idx2, tutor_pad, time_pad, B, E):
    nw = _NC * _NS
    bpw = B // nw
    nch = bpw // _CHUNK
    nrows = B // _CHUNK

    mesh = plsc.VectorSubcoreMesh(
        core_axis_name="c", subcore_axis_name="s",
        num_cores=_NC, num_subcores=_NS)

    @functools.partial(
        pl.kernel,
        mesh=mesh,
        compiler_params=pltpu.CompilerParams(use_tc_tiling_on_sc=True),
        out_type=jax.ShapeDtypeStruct((B, 4 * E), jnp.float32),
        scratch_types=[
            pltpu.VMEM((nch, _CHUNK), jnp.int32),
            pltpu.VMEM((nch, _CHUNK), jnp.int32),
            pltpu.VMEM((bpw, 2 * E), jnp.float32),
            pltpu.VMEM((bpw // 2, 2 * E), jnp.float32),
            pltpu.SemaphoreType.DMA,
            pltpu.SemaphoreType.DMA,
        ],
    )
    def gather_kernel(idx_hbm, ttab_hbm, mtab_hbm, out_hbm,
                      tidx_v, midx_v, trows_v, mrows_v, tsem, msem):
        wid = lax.axis_index("s") * _NC + lax.axis_index("c")
        base = wid * bpw
        half = bpw // 2
        lanes_t = pl.ds(0, 2 * E)
        lanes_m = pl.ds(2 * E, 2 * E)
        pltpu.sync_copy(idx_hbm.at[pl.ds(wid * nch, nch)], tidx_v)
        pltpu.sync_copy(idx_hbm.at[pl.ds(nrows + wid * nch, nch)], midx_v)
        tcopies = []
        for j in range(nch):
            tcopies.append(pltpu.async_copy(
                ttab_hbm.at[tidx_v.at[j]], trows_v.at[pl.ds(j * _CHUNK, _CHUNK)],
                tsem))
        for p in range(2):
            mcopies = []
            for j in range(nch // 2):
                mcopies.append(pltpu.async_copy(
                    mtab_hbm.at[midx_v.at[p * (nch // 2) + j]],
                    mrows_v.at[pl.ds(j * _CHUNK, _CHUNK)], msem))
            for c in mcopies:
                c.wait()
            pltpu.sync_copy(mrows_v, out_hbm.at[pl.ds(base + p * half, half), lanes_m])
        for c in tcopies:
            c.wait()
        pltpu.sync_copy(trows_v, out_hbm.at[pl.ds(base, bpw), lanes_t])

    return gather_kernel(idx2, tutor_pad, time_pad)


def _repack_body(tabT, out):
    t = tabT[...].T
    out[...] = jnp.concatenate(
        [t, jnp.zeros(t.shape, dtype=t.dtype)], axis=1)


def _repack(tabT, bm=2048):
    E, V = tabT.shape
    grid = (pl.cdiv(V, bm),)
    return pl.pallas_call(
        _repack_body,
        grid=grid,
        in_specs=[pl.BlockSpec((E, bm), lambda i: (0, i))],
        out_specs=pl.BlockSpec((bm, 2 * E), lambda i: (i, 0)),
        out_shape=jax.ShapeDtypeStruct((V, 2 * E), jnp.float32),
        compiler_params=pltpu.CompilerParams(
            dimension_semantics=("arbitrary",)),
    )(tabT)


def _mlp_body(emb, featT, w1ab, wblk, bsml, w1, b1, w2, b2, w3, b3, outT):
    f32 = jnp.float32
    small = lax.dot_general(
        featT[...], wblk[...], (((0,), (0,)), ((), ())),
        preferred_element_type=f32) + bsml[...]
    h = (jnp.dot(emb[...], w1ab[...], preferred_element_type=f32)
         + jnp.dot(small, w1[128:224, :], preferred_element_type=f32)
         + b1[...])
    h = jnp.maximum(h, 0.0)
    h = jnp.maximum(jnp.dot(h, w2[...], preferred_element_type=f32) + b2[...], 0.0)
    out = jnp.dot(h, w3[...], preferred_element_type=f32) + b3[...]
    outT[...] = out.T


def _mlp(emb, featT, w1ab, wblk, bsml, W1, b1, W2, b2, W3, b3, bm=2048):
    B = emb.shape[0]
    grid = (B // bm,)
    no = W3.shape[1]

    in_specs = [
        pl.BlockSpec((bm, emb.shape[1]), lambda i: (i, 0)),
        pl.BlockSpec((featT.shape[0], bm), lambda i: (0, i)),
        pl.BlockSpec(w1ab.shape, lambda i: (0, 0)),
        pl.BlockSpec(wblk.shape, lambda i: (0, 0)),
        pl.BlockSpec(bsml.shape, lambda i: (0, 0)),
        pl.BlockSpec(W1.shape, lambda i: (0, 0)),
        pl.BlockSpec(b1.shape, lambda i: (0, 0)),
        pl.BlockSpec(W2.shape, lambda i: (0, 0)),
        pl.BlockSpec(b2.shape, lambda i: (0, 0)),
        pl.BlockSpec(W3.shape, lambda i: (0, 0)),
        pl.BlockSpec(b3.shape, lambda i: (0, 0)),
    ]
    return pl.pallas_call(
        _mlp_body,
        grid=grid,
        in_specs=in_specs,
        out_specs=pl.BlockSpec((no, bm), lambda i: (0, i)),
        out_shape=jax.ShapeDtypeStruct((no, B), jnp.float32),
        compiler_params=pltpu.CompilerParams(
            dimension_semantics=("arbitrary",)),
    )(emb, featT, w1ab, wblk, bsml, W1, b1, W2, b2, W3, b3)


def kernel(tutor_idx, time_idx, experience, subject_pca, grade_pca,
           tutor_table, time_table, Ws, bs, Wg, bg, We, be,
           W1, b1, W2, b2, W3, b3):
    B = tutor_idx.shape[0]
    E = tutor_table.shape[1]

    idx2 = jnp.concatenate([tutor_idx, time_idx]).reshape(2 * B // _CHUNK, _CHUNK)
    tutor_pad = _repack(tutor_table.T)
    time_pad = jnp.pad(time_table, ((0, 0), (0, E)))
    emb = _sc_gather(idx2, tutor_pad, time_pad, B, E)

    featT = jnp.concatenate(
        [subject_pca.T, grade_pca.T, experience[None, :]], axis=0)
    z = jnp.zeros
    f32 = jnp.float32
    wblk = jnp.concatenate([
        jnp.concatenate([Ws, z((10, 64), f32)], axis=1),
        jnp.concatenate([z((5, 32), f32), Wg, z((5, 32), f32)], axis=1),
        jnp.concatenate([z((1, 64), f32), We], axis=1),
    ], axis=0)
    bsml = jnp.concatenate([bs, bg, be])[None, :]
    w1ab = jnp.concatenate([
        W1[0:E, :], z((E, W1.shape[1]), f32),
        W1[E:2 * E, :], z((E, W1.shape[1]), f32),
    ], axis=0)

    outT = _mlp(emb, featT, w1ab, wblk, bsml,
                W1, b1[None, :], W2, b2[None, :], W3, b3[None, :])
    return outT.T


# speedup vs baseline: 2.5277x; 1.2333x over previous
import functools

import jax
import jax.numpy as jnp
from jax import lax
from jax.experimental import pallas as pl
from jax.experimental.pallas import tpu as pltpu
from jax.experimental.pallas import tpu_sc as plsc

_NC = 2
_NS = 16
_CHUNK = 128


def _sc_gather(idx2, tutor_pad, time_pad, B, E):
    nw = _NC * _NS
    bpw = B // nw
    nch = bpw // _CHUNK
    nrows = B // _CHUNK

    mesh = plsc.VectorSubcoreMesh(
        core_axis_name="c", subcore_axis_name="s",
        num_cores=_NC, num_subcores=_NS)

    @functools.partial(
        pl.kernel,
        mesh=mesh,
        compiler_params=pltpu.CompilerParams(use_tc_tiling_on_sc=True),
        out_type=jax.ShapeDtypeStruct((B, 4 * E), jnp.float32),
        scratch_types=[
            pltpu.VMEM((nch, _CHUNK), jnp.int32),
            pltpu.VMEM((nch, _CHUNK), jnp.int32),
            pltpu.VMEM((bpw, 2 * E), jnp.float32),
            pltpu.VMEM((bpw // 2, 2 * E), jnp.float32),
            pltpu.SemaphoreType.DMA,
            pltpu.SemaphoreType.DMA,
        ],
    )
    def gather_kernel(idx_hbm, ttab_hbm, mtab_hbm, out_hbm,
                      tidx_v, midx_v, trows_v, mrows_v, tsem, msem):
        wid = lax.axis_index("s") * _NC + lax.axis_index("c")
        base = wid * bpw
        half = bpw // 2
        lanes_t = pl.ds(0, 2 * E)
        lanes_m = pl.ds(2 * E, 2 * E)
        pltpu.sync_copy(idx_hbm.at[pl.ds(wid * nch, nch)], tidx_v)
        pltpu.sync_copy(idx_hbm.at[pl.ds(nrows + wid * nch, nch)], midx_v)
        tcopies = []
        for j in range(nch):
            tcopies.append(pltpu.async_copy(
                ttab_hbm.at[tidx_v.at[j]], trows_v.at[pl.ds(j * _CHUNK, _CHUNK)],
                tsem))
        for p in range(2):
            mcopies = []
            for j in range(nch // 2):
                mcopies.append(pltpu.async_copy(
                    mtab_hbm.at[midx_v.at[p * (nch // 2) + j]],
                    mrows_v.at[pl.ds(j * _CHUNK, _CHUNK)], msem))
            for c in mcopies:
                c.wait()
            pltpu.sync_copy(mrows_v, out_hbm.at[pl.ds(base + p * half, half), lanes_m])
        for c in tcopies:
            c.wait()
        pltpu.sync_copy(trows_v, out_hbm.at[pl.ds(base, bpw), lanes_t])

    return gather_kernel(idx2, tutor_pad, time_pad)


def _repack_body(tabT, out):
    t = tabT[...].T
    out[...] = jnp.concatenate(
        [t, jnp.zeros(t.shape, dtype=t.dtype)], axis=1)


def _repack(tabT, bm=8192):
    E, V = tabT.shape
    grid = (pl.cdiv(V, bm),)
    return pl.pallas_call(
        _repack_body,
        grid=grid,
        in_specs=[pl.BlockSpec((E, bm), lambda i: (0, i))],
        out_specs=pl.BlockSpec((bm, 2 * E), lambda i: (i, 0)),
        out_shape=jax.ShapeDtypeStruct((V, 2 * E), jnp.float32),
        compiler_params=pltpu.CompilerParams(
            dimension_semantics=("arbitrary",)),
    )(tabT)


def _mlp_body(emb, featT, w1ab, wblk, bsml, w1, b1, w2, b2, w3, b3, outT):
    f32 = jnp.float32
    small = lax.dot_general(
        featT[...], wblk[...], (((0,), (0,)), ((), ())),
        preferred_element_type=f32) + bsml[...]
    h = (jnp.dot(emb[...], w1ab[...], preferred_element_type=f32)
         + jnp.dot(small, w1[128:224, :], preferred_element_type=f32)
         + b1[...])
    h = jnp.maximum(h, 0.0)
    h = jnp.maximum(jnp.dot(h, w2[...], preferred_element_type=f32) + b2[...], 0.0)
    out = jnp.dot(h, w3[...], preferred_element_type=f32) + b3[...]
    outT[...] = out.T


def _mlp(emb, featT, w1ab, wblk, bsml, W1, b1, W2, b2, W3, b3, bm=2048):
    B = emb.shape[0]
    grid = (B // bm,)
    no = W3.shape[1]

    in_specs = [
        pl.BlockSpec((bm, emb.shape[1]), lambda i: (i, 0)),
        pl.BlockSpec((featT.shape[0], bm), lambda i: (0, i)),
        pl.BlockSpec(w1ab.shape, lambda i: (0, 0)),
        pl.BlockSpec(wblk.shape, lambda i: (0, 0)),
        pl.BlockSpec(bsml.shape, lambda i: (0, 0)),
        pl.BlockSpec(W1.shape, lambda i: (0, 0)),
        pl.BlockSpec(b1.shape, lambda i: (0, 0)),
        pl.BlockSpec(W2.shape, lambda i: (0, 0)),
        pl.BlockSpec(b2.shape, lambda i: (0, 0)),
        pl.BlockSpec(W3.shape, lambda i: (0, 0)),
        pl.BlockSpec(b3.shape, lambda i: (0, 0)),
    ]
    return pl.pallas_call(
        _mlp_body,
        grid=grid,
        in_specs=in_specs,
        out_specs=pl.BlockSpec((no, bm), lambda i: (0, i)),
        out_shape=jax.ShapeDtypeStruct((no, B), jnp.float32),
        compiler_params=pltpu.CompilerParams(
            dimension_semantics=("arbitrary",)),
    )(emb, featT, w1ab, wblk, bsml, W1, b1, W2, b2, W3, b3)


def kernel(tutor_idx, time_idx, experience, subject_pca, grade_pca,
           tutor_table, time_table, Ws, bs, Wg, bg, We, be,
           W1, b1, W2, b2, W3, b3):
    B = tutor_idx.shape[0]
    E = tutor_table.shape[1]

    idx2 = jnp.concatenate([tutor_idx, time_idx]).reshape(2 * B // _CHUNK, _CHUNK)
    tutor_pad = _repack(tutor_table.T)
    time_pad = jnp.pad(time_table, ((0, 0), (0, E)))
    emb = _sc_gather(idx2, tutor_pad, time_pad, B, E)

    featT = jnp.concatenate(
        [subject_pca.T, grade_pca.T, experience[None, :]], axis=0)
    z = jnp.zeros
    f32 = jnp.float32
    wblk = jnp.concatenate([
        jnp.concatenate([Ws, z((10, 64), f32)], axis=1),
        jnp.concatenate([z((5, 32), f32), Wg, z((5, 32), f32)], axis=1),
        jnp.concatenate([z((1, 64), f32), We], axis=1),
    ], axis=0)
    bsml = jnp.concatenate([bs, bg, be])[None, :]
    w1ab = jnp.concatenate([
        W1[0:E, :], z((E, W1.shape[1]), f32),
        W1[E:2 * E, :], z((E, W1.shape[1]), f32),
    ], axis=0)

    outT = _mlp(emb, featT, w1ab, wblk, bsml,
                W1, b1[None, :], W2, b2[None, :], W3, b3[None, :])
    return outT.T


# speedup vs baseline: 2.5557x; 1.0111x over previous
import functools

import jax
import jax.numpy as jnp
from jax import lax
from jax.experimental import pallas as pl
from jax.experimental.pallas import tpu as pltpu
from jax.experimental.pallas import tpu_sc as plsc

_NC = 2
_NS = 16
_CHUNK = 128


def _sc_gather(idx2, tutor_pad, time_pad, B, E):
    nw = _NC * _NS
    bpw = B // nw
    nch = bpw // _CHUNK
    nrows = B // _CHUNK

    mesh = plsc.VectorSubcoreMesh(
        core_axis_name="c", subcore_axis_name="s",
        num_cores=_NC, num_subcores=_NS)

    @functools.partial(
        pl.kernel,
        mesh=mesh,
        compiler_params=pltpu.CompilerParams(use_tc_tiling_on_sc=True),
        out_type=jax.ShapeDtypeStruct((B, 4 * E), jnp.float32),
        scratch_types=[
            pltpu.VMEM((nch, _CHUNK), jnp.int32),
            pltpu.VMEM((nch, _CHUNK), jnp.int32),
            pltpu.VMEM((bpw, 2 * E), jnp.float32),
            pltpu.VMEM((bpw // 2, 2 * E), jnp.float32),
            pltpu.SemaphoreType.DMA,
            pltpu.SemaphoreType.DMA,
        ],
    )
    def gather_kernel(idx_hbm, ttab_hbm, mtab_hbm, out_hbm,
                      tidx_v, midx_v, trows_v, mrows_v, tsem, msem):
        wid = lax.axis_index("s") * _NC + lax.axis_index("c")
        base = wid * bpw
        half = bpw // 2
        lanes_t = pl.ds(0, 2 * E)
        lanes_m = pl.ds(2 * E, 2 * E)
        pltpu.sync_copy(idx_hbm.at[pl.ds(wid * nch, nch)], tidx_v)
        pltpu.sync_copy(idx_hbm.at[pl.ds(nrows + wid * nch, nch)], midx_v)
        tcopies = []
        for j in range(nch):
            tcopies.append(pltpu.async_copy(
                ttab_hbm.at[tidx_v.at[j]], trows_v.at[pl.ds(j * _CHUNK, _CHUNK)],
                tsem))
        for p in range(2):
            mcopies = []
            for j in range(nch // 2):
                mcopies.append(pltpu.async_copy(
                    mtab_hbm.at[midx_v.at[p * (nch // 2) + j]],
                    mrows_v.at[pl.ds(j * _CHUNK, _CHUNK)], msem))
            for c in mcopies:
                c.wait()
            pltpu.sync_copy(mrows_v, out_hbm.at[pl.ds(base + p * half, half), lanes_m])
        for c in tcopies:
            c.wait()
        pltpu.sync_copy(trows_v, out_hbm.at[pl.ds(base, bpw), lanes_t])

    return gather_kernel(idx2, tutor_pad, time_pad)


def _repack_body(tabT, out):
    t = tabT[...].T
    out[...] = jnp.concatenate(
        [t, jnp.zeros(t.shape, dtype=t.dtype)], axis=1)


def _repack(tabT, bm=16384):
    E, V = tabT.shape
    grid = (pl.cdiv(V, bm),)
    return pl.pallas_call(
        _repack_body,
        grid=grid,
        in_specs=[pl.BlockSpec((E, bm), lambda i: (0, i))],
        out_specs=pl.BlockSpec((bm, 2 * E), lambda i: (i, 0)),
        out_shape=jax.ShapeDtypeStruct((V, 2 * E), jnp.float32),
        compiler_params=pltpu.CompilerParams(
            dimension_semantics=("arbitrary",)),
    )(tabT)


def _mlp_body(emb, featT, w1ab, wblk, bsml, w1, b1, w2, b2, w3, b3, outT):
    f32 = jnp.float32
    small = lax.dot_general(
        featT[...], wblk[...], (((0,), (0,)), ((), ())),
        preferred_element_type=f32) + bsml[...]
    h = (jnp.dot(emb[...], w1ab[...], preferred_element_type=f32)
         + jnp.dot(small, w1[128:224, :], preferred_element_type=f32)
         + b1[...])
    h = jnp.maximum(h, 0.0)
    h = jnp.maximum(jnp.dot(h, w2[...], preferred_element_type=f32) + b2[...], 0.0)
    out = jnp.dot(h, w3[...], preferred_element_type=f32) + b3[...]
    outT[...] = out.T


def _mlp(emb, featT, w1ab, wblk, bsml, W1, b1, W2, b2, W3, b3, bm=2048):
    B = emb.shape[0]
    grid = (B // bm,)
    no = W3.shape[1]

    in_specs = [
        pl.BlockSpec((bm, emb.shape[1]), lambda i: (i, 0)),
        pl.BlockSpec((featT.shape[0], bm), lambda i: (0, i)),
        pl.BlockSpec(w1ab.shape, lambda i: (0, 0)),
        pl.BlockSpec(wblk.shape, lambda i: (0, 0)),
        pl.BlockSpec(bsml.shape, lambda i: (0, 0)),
        pl.BlockSpec(W1.shape, lambda i: (0, 0)),
        pl.BlockSpec(b1.shape, lambda i: (0, 0)),
        pl.BlockSpec(W2.shape, lambda i: (0, 0)),
        pl.BlockSpec(b2.shape, lambda i: (0, 0)),
        pl.BlockSpec(W3.shape, lambda i: (0, 0)),
        pl.BlockSpec(b3.shape, lambda i: (0, 0)),
    ]
    return pl.pallas_call(
        _mlp_body,
        grid=grid,
        in_specs=in_specs,
        out_specs=pl.BlockSpec((no, bm), lambda i: (0, i)),
        out_shape=jax.ShapeDtypeStruct((no, B), jnp.float32),
        compiler_params=pltpu.CompilerParams(
            dimension_semantics=("arbitrary",)),
    )(emb, featT, w1ab, wblk, bsml, W1, b1, W2, b2, W3, b3)


def kernel(tutor_idx, time_idx, experience, subject_pca, grade_pca,
           tutor_table, time_table, Ws, bs, Wg, bg, We, be,
           W1, b1, W2, b2, W3, b3):
    B = tutor_idx.shape[0]
    E = tutor_table.shape[1]

    idx2 = jnp.concatenate([tutor_idx, time_idx]).reshape(2 * B // _CHUNK, _CHUNK)
    tutor_pad = _repack(tutor_table.T)
    time_pad = jnp.pad(time_table, ((0, 0), (0, E)))
    emb = _sc_gather(idx2, tutor_pad, time_pad, B, E)

    featT = jnp.concatenate(
        [subject_pca.T, grade_pca.T, experience[None, :]], axis=0)
    z = jnp.zeros
    f32 = jnp.float32
    wblk = jnp.concatenate([
        jnp.concatenate([Ws, z((10, 64), f32)], axis=1),
        jnp.concatenate([z((5, 32), f32), Wg, z((5, 32), f32)], axis=1),
        jnp.concatenate([z((1, 64), f32), We], axis=1),
    ], axis=0)
    bsml = jnp.concatenate([bs, bg, be])[None, :]
    w1ab = jnp.concatenate([
        W1[0:E, :], z((E, W1.shape[1]), f32),
        W1[E:2 * E, :], z((E, W1.shape[1]), f32),
    ], axis=0)

    outT = _mlp(emb, featT, w1ab, wblk, bsml,
                W1, b1[None, :], W2, b2[None, :], W3, b3[None, :])
    return outT.T


# speedup vs baseline: 2.5613x; 1.0022x over previous
import functools

import jax
import jax.numpy as jnp
from jax import lax
from jax.experimental import pallas as pl
from jax.experimental.pallas import tpu as pltpu
from jax.experimental.pallas import tpu_sc as plsc

_NC = 2
_NS = 16
_CHUNK = 128


def _sc_gather(idx2, tutor_pad, time_pad, B, E):
    nw = _NC * _NS
    bpw = B // nw
    nch = bpw // _CHUNK
    nrows = B // _CHUNK

    mesh = plsc.VectorSubcoreMesh(
        core_axis_name="c", subcore_axis_name="s",
        num_cores=_NC, num_subcores=_NS)

    @functools.partial(
        pl.kernel,
        mesh=mesh,
        compiler_params=pltpu.CompilerParams(use_tc_tiling_on_sc=True),
        out_type=jax.ShapeDtypeStruct((B, 4 * E), jnp.float32),
        scratch_types=[
            pltpu.VMEM((nch, _CHUNK), jnp.int32),
            pltpu.VMEM((nch, _CHUNK), jnp.int32),
            pltpu.VMEM((bpw, 2 * E), jnp.float32),
            pltpu.VMEM((bpw // 2, 2 * E), jnp.float32),
            pltpu.SemaphoreType.DMA,
            pltpu.SemaphoreType.DMA,
        ],
    )
    def gather_kernel(idx_hbm, ttab_hbm, mtab_hbm, out_hbm,
                      tidx_v, midx_v, trows_v, mrows_v, tsem, msem):
        wid = lax.axis_index("s") * _NC + lax.axis_index("c")
        base = wid * bpw
        half = bpw // 2
        lanes_t = pl.ds(0, 2 * E)
        lanes_m = pl.ds(2 * E, 2 * E)
        pltpu.sync_copy(idx_hbm.at[pl.ds(wid * nch, nch)], tidx_v)
        pltpu.sync_copy(idx_hbm.at[pl.ds(nrows + wid * nch, nch)], midx_v)
        tcopies = []
        for j in range(nch):
            tcopies.append(pltpu.async_copy(
                ttab_hbm.at[tidx_v.at[j]], trows_v.at[pl.ds(j * _CHUNK, _CHUNK)],
                tsem))
        for p in range(2):
            mcopies = []
            for j in range(nch // 2):
                mcopies.append(pltpu.async_copy(
                    mtab_hbm.at[midx_v.at[p * (nch // 2) + j]],
                    mrows_v.at[pl.ds(j * _CHUNK, _CHUNK)], msem))
            for c in mcopies:
                c.wait()
            pltpu.sync_copy(mrows_v, out_hbm.at[pl.ds(base + p * half, half), lanes_m])
        for c in tcopies:
            c.wait()
        pltpu.sync_copy(trows_v, out_hbm.at[pl.ds(base, bpw), lanes_t])

    return gather_kernel(idx2, tutor_pad, time_pad)


def _repack_body(tabT, out):
    t = tabT[...].T
    out[...] = jnp.concatenate(
        [t, jnp.zeros(t.shape, dtype=t.dtype)], axis=1)


def _repack(tabT, bm=16384):
    E, V = tabT.shape
    grid = (pl.cdiv(V, bm),)
    return pl.pallas_call(
        _repack_body,
        grid=grid,
        in_specs=[pl.BlockSpec((E, bm), lambda i: (0, i))],
        out_specs=pl.BlockSpec((bm, 2 * E), lambda i: (i, 0)),
        out_shape=jax.ShapeDtypeStruct((V, 2 * E), jnp.float32),
        compiler_params=pltpu.CompilerParams(
            dimension_semantics=("arbitrary",)),
    )(tabT)


def _mlp_body(emb, featT, w1ab, wblk, bsml, w1, b1, w2, b2, w3, b3, outT):
    f32 = jnp.float32
    bf16 = jnp.bfloat16
    small = lax.dot_general(
        featT[...], wblk[...], (((0,), (0,)), ((), ())),
        preferred_element_type=f32) + bsml[...]
    h = (jnp.dot(emb[...].astype(bf16), w1ab[...].astype(bf16),
                 preferred_element_type=f32)
         + jnp.dot(small.astype(bf16), w1[128:224, :].astype(bf16),
                   preferred_element_type=f32)
         + b1[...])
    h = jnp.maximum(h, 0.0)
    h = jnp.maximum(
        jnp.dot(h.astype(bf16), w2[...].astype(bf16),
                preferred_element_type=f32) + b2[...], 0.0)
    out = jnp.dot(h.astype(bf16), w3[...].astype(bf16),
                  preferred_element_type=f32) + b3[...]
    outT[...] = out.T


def _mlp(emb, featT, w1ab, wblk, bsml, W1, b1, W2, b2, W3, b3, bm=2048):
    B = emb.shape[0]
    grid = (B // bm,)
    no = W3.shape[1]

    in_specs = [
        pl.BlockSpec((bm, emb.shape[1]), lambda i: (i, 0)),
        pl.BlockSpec((featT.shape[0], bm), lambda i: (0, i)),
        pl.BlockSpec(w1ab.shape, lambda i: (0, 0)),
        pl.BlockSpec(wblk.shape, lambda i: (0, 0)),
        pl.BlockSpec(bsml.shape, lambda i: (0, 0)),
        pl.BlockSpec(W1.shape, lambda i: (0, 0)),
        pl.BlockSpec(b1.shape, lambda i: (0, 0)),
        pl.BlockSpec(W2.shape, lambda i: (0, 0)),
        pl.BlockSpec(b2.shape, lambda i: (0, 0)),
        pl.BlockSpec(W3.shape, lambda i: (0, 0)),
        pl.BlockSpec(b3.shape, lambda i: (0, 0)),
    ]
    return pl.pallas_call(
        _mlp_body,
        grid=grid,
        in_specs=in_specs,
        out_specs=pl.BlockSpec((no, bm), lambda i: (0, i)),
        out_shape=jax.ShapeDtypeStruct((no, B), jnp.float32),
        compiler_params=pltpu.CompilerParams(
            dimension_semantics=("arbitrary",)),
    )(emb, featT, w1ab, wblk, bsml, W1, b1, W2, b2, W3, b3)


def kernel(tutor_idx, time_idx, experience, subject_pca, grade_pca,
           tutor_table, time_table, Ws, bs, Wg, bg, We, be,
           W1, b1, W2, b2, W3, b3):
    B = tutor_idx.shape[0]
    E = tutor_table.shape[1]

    idx2 = jnp.concatenate([tutor_idx, time_idx]).reshape(2 * B // _CHUNK, _CHUNK)
    tutor_pad = _repack(tutor_table.T)
    time_pad = jnp.pad(time_table, ((0, 0), (0, E)))
    emb = _sc_gather(idx2, tutor_pad, time_pad, B, E)

    featT = jnp.concatenate(
        [subject_pca.T, grade_pca.T, experience[None, :]], axis=0)
    z = jnp.zeros
    f32 = jnp.float32
    wblk = jnp.concatenate([
        jnp.concatenate([Ws, z((10, 64), f32)], axis=1),
        jnp.concatenate([z((5, 32), f32), Wg, z((5, 32), f32)], axis=1),
        jnp.concatenate([z((1, 64), f32), We], axis=1),
    ], axis=0)
    bsml = jnp.concatenate([bs, bg, be])[None, :]
    w1ab = jnp.concatenate([
        W1[0:E, :], z((E, W1.shape[1]), f32),
        W1[E:2 * E, :], z((E, W1.shape[1]), f32),
    ], axis=0)

    outT = _mlp(emb, featT, w1ab, wblk, bsml,
                W1, b1[None, :], W2, b2[None, :], W3, b3[None, :])
    return outT.T
